# initial kernel scaffold (unmeasured)
import jax
import jax.numpy as jnp
from jax import lax
from jax.experimental import pallas as pl
from jax.experimental.pallas import tpu as pltpu

N_Z = 4


def kernel(x, dy):
    k, m = x.shape
    _, n = dy.shape
    m_out = m // N_Z

    x_bf = x.astype(jnp.bfloat16)
    dy_bf = dy.astype(jnp.bfloat16)

    def body(x_ref, dy_ref, out_ref, send_buf, recv_bufs, send_sem, recv_sems):
        my_x = lax.axis_index("x")
        my_y = lax.axis_index("y")
        my_z = lax.axis_index("z")
        right = (my_z + 1) % N_Z
        left = (my_z + N_Z - 1) % N_Z

        barrier = pltpu.get_barrier_semaphore()
        for nbr in (left, right):
            pl.semaphore_signal(
                barrier,
                inc=1,
                device_id=(my_x, my_y, nbr),
                device_id_type=pl.DeviceIdType.MESH,
            )
        pl.semaphore_wait(barrier, 2)

        def partial_chunk(c):
            xs = x_ref[:, pl.ds(c * m_out, m_out)]
            return lax.dot_general(
                xs,
                dy_ref[:, :],
                (((0,), (0,)), ((), ())),
                preferred_element_type=jnp.float32,
            )

        c0 = (my_z + N_Z - 1) % N_Z
        out_ref[:, :] = partial_chunk(c0)

        for s in range(N_Z - 1):
            send_buf[:, :] = out_ref[:, :].astype(jnp.bfloat16)
            rdma = pltpu.make_async_remote_copy(
                src_ref=send_buf,
                dst_ref=recv_bufs.at[s],
                send_sem=send_sem,
                recv_sem=recv_sems.at[s],
                device_id=(my_x, my_y, right),
                device_id_type=pl.DeviceIdType.MESH,
            )
            rdma.start()
            rdma.wait_send()
            rdma.wait_recv()
            c = (my_z + 2 * N_Z - 2 - s) % N_Z
            out_ref[:, :] = recv_bufs[s].astype(jnp.float32) + partial_chunk(c)

    return pl.pallas_call(
        body,
        out_shape=jax.ShapeDtypeStruct((m_out, n), jnp.float32),
        in_specs=[
            pl.BlockSpec(memory_space=pltpu.VMEM),
            pl.BlockSpec(memory_space=pltpu.VMEM),
        ],
        out_specs=pl.BlockSpec(memory_space=pltpu.VMEM),
        scratch_shapes=[
            pltpu.VMEM((m_out, n), jnp.bfloat16),
            pltpu.VMEM((N_Z - 1, m_out, n), jnp.bfloat16),
            pltpu.SemaphoreType.DMA,
            pltpu.SemaphoreType.DMA((N_Z - 1,)),
        ],
        compiler_params=pltpu.CompilerParams(collective_id=0),
    )(x_bf, dy_bf)


# baseline (device time: 442160 ns/iter reference)
import jax
import jax.numpy as jnp
from jax import lax
from jax.experimental import pallas as pl
from jax.experimental.pallas import tpu as pltpu

N_Z = 4
NT = 512


def kernel(x, dy):
    k, m = x.shape
    _, n = dy.shape
    m_out = m // N_Z
    n_tiles = n // NT

    x_bf = x.astype(jnp.bfloat16)
    dy_bf = dy.astype(jnp.bfloat16)

    def body(x_ref, dy_hbm, out_ref, send_buf, recv_bufs, dy_tiles,
             send_sem, recv_sems, dy_sems):
        my_x = lax.axis_index("x")
        my_y = lax.axis_index("y")
        my_z = lax.axis_index("z")
        right = (my_z + 1) % N_Z
        left = (my_z + N_Z - 1) % N_Z

        barrier = pltpu.get_barrier_semaphore()
        for nbr in (left, right):
            pl.semaphore_signal(
                barrier,
                inc=1,
                device_id=(my_x, my_y, nbr),
                device_id_type=pl.DeviceIdType.MESH,
            )
        pl.semaphore_wait(barrier, 2)

        def dy_dma(j, slot):
            return pltpu.make_async_copy(
                dy_hbm.at[:, pl.ds(j * NT, NT)],
                dy_tiles.at[slot],
                dy_sems.at[slot],
            )

        def compute_chunk(c, s):
            xs = x_ref[:, pl.ds(c * m_out, m_out)]
            dy_dma(0, 0).start()
            for j in range(n_tiles):
                slot = j % 2
                if j + 1 < n_tiles:
                    dy_dma(j + 1, (j + 1) % 2).start()
                dy_dma(j, slot).wait()
                p = lax.dot_general(
                    xs,
                    dy_tiles[slot],
                    (((0,), (0,)), ((), ())),
                    preferred_element_type=jnp.float32,
                )
                cols = pl.ds(j * NT, NT)
                if s is None:
                    out_ref[:, cols] = p
                else:
                    out_ref[:, cols] = recv_bufs[s, :, cols].astype(jnp.float32) + p

        compute_chunk((my_z + N_Z - 1) % N_Z, None)

        for s in range(N_Z - 1):
            send_buf[:, :] = out_ref[:, :].astype(jnp.bfloat16)
            rdma = pltpu.make_async_remote_copy(
                src_ref=send_buf,
                dst_ref=recv_bufs.at[s],
                send_sem=send_sem,
                recv_sem=recv_sems.at[s],
                device_id=(my_x, my_y, right),
                device_id_type=pl.DeviceIdType.MESH,
            )
            rdma.start()
            rdma.wait_send()
            rdma.wait_recv()
            compute_chunk((my_z + 2 * N_Z - 2 - s) % N_Z, s)

    return pl.pallas_call(
        body,
        out_shape=jax.ShapeDtypeStruct((m_out, n), jnp.float32),
        in_specs=[
            pl.BlockSpec(memory_space=pltpu.VMEM),
            pl.BlockSpec(memory_space=pltpu.MemorySpace.HBM),
        ],
        out_specs=pl.BlockSpec(memory_space=pltpu.VMEM),
        scratch_shapes=[
            pltpu.VMEM((m_out, n), jnp.bfloat16),
            pltpu.VMEM((N_Z - 1, m_out, n), jnp.bfloat16),
            pltpu.VMEM((2, k, NT), jnp.bfloat16),
            pltpu.SemaphoreType.DMA,
            pltpu.SemaphoreType.DMA((N_Z - 1,)),
            pltpu.SemaphoreType.DMA((2,)),
        ],
        compiler_params=pltpu.CompilerParams(
            collective_id=0,
            vmem_limit_bytes=64 * 1024 * 1024,
        ),
    )(x_bf, dy_bf)


# device time: 371901 ns/iter; 1.1889x vs baseline; 1.1889x over previous
import jax
import jax.numpy as jnp
from jax import lax
from jax.experimental import pallas as pl
from jax.experimental.pallas import tpu as pltpu

N_Z = 4
NT = 512


def kernel(x, dy):
    k, m = x.shape
    _, n = dy.shape
    m_out = m // N_Z
    n_tiles = n // NT

    x_bf = x.astype(jnp.bfloat16)
    dy_bf = dy.astype(jnp.bfloat16)

    def body(x_ref, dy_hbm, out_ref, s_bufs, r_bufs, dy_tiles,
             send_sems, recv_sems, credit_sem, dy_sems):
        my_x = lax.axis_index("x")
        my_y = lax.axis_index("y")
        my_z = lax.axis_index("z")
        right = (my_z + 1) % N_Z
        left = (my_z + N_Z - 1) % N_Z

        def dy_dma(j, slot):
            return pltpu.make_async_copy(
                dy_hbm.at[:, pl.ds(j * NT, NT)],
                dy_tiles.at[slot],
                dy_sems.at[slot],
            )

        def compute_partial(c, dst):
            xs = x_ref[:, pl.ds(c * m_out, m_out)]
            dy_dma(0, 0).start()
            for j in range(n_tiles):
                slot = j % 2
                if j + 1 < n_tiles:
                    dy_dma(j + 1, (j + 1) % 2).start()
                dy_dma(j, slot).wait()
                p = lax.dot_general(
                    xs,
                    dy_tiles[slot],
                    (((0,), (0,)), ((), ())),
                    preferred_element_type=jnp.float32,
                )
                dst[:, pl.ds(j * NT, NT)] = p.astype(jnp.bfloat16)

        def hop(s, src_slot, dst_slot):
            return pltpu.make_async_remote_copy(
                src_ref=s_bufs.at[src_slot],
                dst_ref=r_bufs.at[dst_slot],
                send_sem=send_sems.at[s],
                recv_sem=recv_sems.at[s],
                device_id=(my_x, my_y, right),
                device_id_type=pl.DeviceIdType.MESH,
            )

        compute_partial((my_z + N_Z - 1) % N_Z, s_bufs.at[0])

        barrier = pltpu.get_barrier_semaphore()
        for nbr in (left, right):
            pl.semaphore_signal(
                barrier,
                inc=1,
                device_id=(my_x, my_y, nbr),
                device_id_type=pl.DeviceIdType.MESH,
            )
        pl.semaphore_wait(barrier, 2)

        hop0 = hop(0, 0, 0)
        hop0.start()
        compute_partial((my_z + N_Z - 2) % N_Z, s_bufs.at[1])
        hop0.wait_recv()
        s_bufs[1, :, :] += r_bufs[0, :, :]
        pl.semaphore_signal(
            credit_sem,
            inc=1,
            device_id=(my_x, my_y, left),
            device_id_type=pl.DeviceIdType.MESH,
        )

        hop1 = hop(1, 1, 1)
        hop1.start()
        hop0.wait_send()
        compute_partial((my_z + N_Z - 3) % N_Z, s_bufs.at[0])
        hop1.wait_recv()
        s_bufs[0, :, :] += r_bufs[1, :, :]

        pl.semaphore_wait(credit_sem, 1)
        hop2 = hop(2, 0, 0)
        hop2.start()
        hop1.wait_send()
        compute_partial(my_z, s_bufs.at[1])
        hop2.wait_recv()
        out_ref[:, :] = (
            s_bufs[1, :, :].astype(jnp.float32)
            + r_bufs[0, :, :].astype(jnp.float32)
        )
        hop2.wait_send()

    return pl.pallas_call(
        body,
        out_shape=jax.ShapeDtypeStruct((m_out, n), jnp.float32),
        in_specs=[
            pl.BlockSpec(memory_space=pltpu.VMEM),
            pl.BlockSpec(memory_space=pltpu.MemorySpace.HBM),
        ],
        out_specs=pl.BlockSpec(memory_space=pltpu.VMEM),
        scratch_shapes=[
            pltpu.VMEM((2, m_out, n), jnp.bfloat16),
            pltpu.VMEM((2, m_out, n), jnp.bfloat16),
            pltpu.VMEM((2, k, NT), jnp.bfloat16),
            pltpu.SemaphoreType.DMA((N_Z - 1,)),
            pltpu.SemaphoreType.DMA((N_Z - 1,)),
            pltpu.SemaphoreType.REGULAR,
            pltpu.SemaphoreType.DMA((2,)),
        ],
        compiler_params=pltpu.CompilerParams(
            collective_id=0,
            vmem_limit_bytes=64 * 1024 * 1024,
        ),
    )(x_bf, dy_bf)
